# baseline (speedup 1.0000x reference)
import functools
import math

import jax
import jax.numpy as jnp
from jax import lax
from jax.experimental import pallas as pl
from jax.experimental.pallas import tpu as pltpu
from jax.experimental.pallas import tpu_sc as plsc

_VOCAB = 100000
_HID = 1024
_BSZ = 4
_QLEN = 2048
_NIDS = _BSZ * _QLEN
_NW = 32
_IDS_PER_W = _NIDS // _NW
_CHUNK = 32
_NCHUNK = _IDS_PER_W // _CHUNK


def _gather_body(ids_hbm, table_hbm, out_hbm, idx_v, buf_a, buf_b, sem_g, sem_s):
    wid = lax.axis_index("s") * 2 + lax.axis_index("c")
    base = wid * _IDS_PER_W
    pltpu.sync_copy(ids_hbm.at[wid], idx_v)

    bufs = [buf_a, buf_b]
    gathers = [None] * _NCHUNK
    stores = [None] * _NCHUNK
    gathers[0] = pltpu.async_copy(table_hbm.at[idx_v.at[0]], bufs[0], sem_g)
    for c in range(_NCHUNK):
        if c + 1 < _NCHUNK:
            if c >= 1:
                stores[c - 1].wait()
            gathers[c + 1] = pltpu.async_copy(
                table_hbm.at[idx_v.at[c + 1]], bufs[(c + 1) % 2], sem_g)
        gathers[c].wait()
        stores[c] = pltpu.async_copy(
            bufs[c % 2], out_hbm.at[pl.ds(base + c * _CHUNK, _CHUNK)], sem_s)
    stores[_NCHUNK - 2].wait()
    stores[_NCHUNK - 1].wait()


@functools.partial(jax.jit, static_argnums=())
def _sc_gather(ids_flat, table):
    mesh = plsc.VectorSubcoreMesh(core_axis_name="c", subcore_axis_name="s")
    run = pl.kernel(
        _gather_body,
        out_type=jax.ShapeDtypeStruct((_NIDS, _HID), jnp.float32),
        mesh=mesh,
        scratch_types=[
            pltpu.VMEM((_NCHUNK, _CHUNK), jnp.int32),
            pltpu.VMEM((_CHUNK, _HID), jnp.float32),
            pltpu.VMEM((_CHUNK, _HID), jnp.float32),
            pltpu.SemaphoreType.DMA,
            pltpu.SemaphoreType.DMA,
        ],
    )
    return run(ids_flat.reshape(_NW, _NCHUNK, _CHUNK), table)


_NPE = 2 * _QLEN
_HALF = _HID // 2
_PBR = 512


def _pos_body(o_ref):
    i = pl.program_id(0)
    p = lax.broadcasted_iota(jnp.int32, (_PBR, _HALF), 0) + i * _PBR
    c = lax.broadcasted_iota(jnp.int32, (_PBR, _HALF), 1)
    s = (_QLEN - p).astype(jnp.float32)
    inv_freq = jnp.exp(c.astype(jnp.float32) * (-math.log(10000.0) / _HALF))
    angle = s * inv_freq
    pe = jnp.concatenate([jnp.sin(angle), jnp.cos(angle)], axis=1)
    for j in range(_BSZ):
        o_ref[:, j * _HID:(j + 1) * _HID] = pe


def _pos_emb():
    out = pl.pallas_call(
        _pos_body,
        out_shape=jax.ShapeDtypeStruct((_NPE, _BSZ * _HID), jnp.float32),
        grid=(_NPE // _PBR,),
        out_specs=pl.BlockSpec((_PBR, _BSZ * _HID), lambda i: (i, 0)),
    )()
    return out.reshape(_BSZ, 2 * _QLEN, _HID)


def kernel(token_id_input, W):
    word = _sc_gather(token_id_input.reshape(-1), W)
    pos = _pos_emb()
    return (word.reshape(_BSZ, _QLEN, _HID), pos)


# speedup vs baseline: 1.8399x; 1.8399x over previous
import functools
import math

import jax
import jax.numpy as jnp
import numpy as np
from jax import lax
from jax.experimental import pallas as pl
from jax.experimental.pallas import tpu as pltpu
from jax.experimental.pallas import tpu_sc as plsc

_VOCAB = 100000
_HID = 1024
_BSZ = 4
_QLEN = 2048
_NIDS = _BSZ * _QLEN
_NW = 32
_IDS_PER_W = _NIDS // _NW
_CHUNK = 32
_NCHUNK = _IDS_PER_W // _CHUNK


def _gather_body(ids_hbm, table_hbm, out_hbm, idx_v, buf_a, buf_b, sem_g, sem_s):
    wid = lax.axis_index("s") * 2 + lax.axis_index("c")
    base = wid * _IDS_PER_W
    pltpu.sync_copy(ids_hbm.at[wid], idx_v)

    bufs = [buf_a, buf_b]
    gathers = [None] * _NCHUNK
    stores = [None] * _NCHUNK
    gathers[0] = pltpu.async_copy(table_hbm.at[idx_v.at[0]], bufs[0], sem_g)
    for c in range(_NCHUNK):
        if c + 1 < _NCHUNK:
            if c >= 1:
                stores[c - 1].wait()
            gathers[c + 1] = pltpu.async_copy(
                table_hbm.at[idx_v.at[c + 1]], bufs[(c + 1) % 2], sem_g)
        gathers[c].wait()
        stores[c] = pltpu.async_copy(
            bufs[c % 2], out_hbm.at[pl.ds(base + c * _CHUNK, _CHUNK)], sem_s)
    stores[_NCHUNK - 2].wait()
    stores[_NCHUNK - 1].wait()


@functools.partial(jax.jit, static_argnums=())
def _sc_gather(ids_flat, table):
    mesh = plsc.VectorSubcoreMesh(core_axis_name="c", subcore_axis_name="s")
    run = pl.kernel(
        _gather_body,
        out_type=jax.ShapeDtypeStruct((_NIDS, _HID), jnp.float32),
        mesh=mesh,
        scratch_types=[
            pltpu.VMEM((_NCHUNK, _CHUNK), jnp.int32),
            pltpu.VMEM((_CHUNK, _HID), jnp.float32),
            pltpu.VMEM((_CHUNK, _HID), jnp.float32),
            pltpu.SemaphoreType.DMA,
            pltpu.SemaphoreType.DMA,
        ],
    )
    return run(ids_flat.reshape(_NW, _NCHUNK, _CHUNK), table)


_NPE = 2 * _QLEN
_HALF = _HID // 2
_UR2 = 128

_INV_FREQ = np.exp(np.arange(_HALF, dtype=np.float64)
                   * (-math.log(10000.0) / _HALF)).astype(np.float32)


def _pos_body(invf_ref, o_ref):
    i = pl.program_id(0)
    u = lax.broadcasted_iota(jnp.int32, (_UR2, _HALF), 0).astype(jnp.float32)
    inv_freq = jnp.broadcast_to(invf_ref[0:1, :], (_UR2, _HALF))
    s_even = (_QLEN - 2 * _UR2 * i) - 2.0 * u
    ang_e = s_even * inv_freq
    ang_o = ang_e - inv_freq
    pe_e = jnp.concatenate([jnp.sin(ang_e), jnp.cos(ang_e)], axis=1)
    pe_o = jnp.concatenate([jnp.sin(ang_o), jnp.cos(ang_o)], axis=1)
    sub = lax.broadcasted_iota(jnp.int32, (_UR2, 8, _HID), 1)
    o_ref[...] = jnp.where(sub < 4, pe_e[:, None, :], pe_o[:, None, :])


def _pos_emb():
    invf = jnp.asarray(np.broadcast_to(_INV_FREQ[None, :], (8, _HALF)))
    out = pl.pallas_call(
        _pos_body,
        out_shape=jax.ShapeDtypeStruct((_NPE * _BSZ // 8, 8, _HID), jnp.float32),
        grid=(_NPE // (2 * _UR2),),
        in_specs=[pl.BlockSpec((8, _HALF), lambda i: (0, 0))],
        out_specs=pl.BlockSpec((_UR2, 8, _HID), lambda i: (i, 0, 0)),
    )(invf)
    return out.reshape(_BSZ, 2 * _QLEN, _HID)


def kernel(token_id_input, W):
    word = _sc_gather(token_id_input.reshape(-1), W)
    pos = _pos_emb()
    return (word.reshape(_BSZ, _QLEN, _HID), pos)


# speedup vs baseline: 1.9417x; 1.0553x over previous
import functools
import math

import jax
import jax.numpy as jnp
import numpy as np
from jax import lax
from jax.experimental import pallas as pl
from jax.experimental.pallas import tpu as pltpu
from jax.experimental.pallas import tpu_sc as plsc

_VOCAB = 100000
_HID = 1024
_BSZ = 4
_QLEN = 2048
_NIDS = _BSZ * _QLEN
_NW = 32
_IDS_PER_W = _NIDS // _NW
_CHUNK = 32
_NCHUNK = _IDS_PER_W // _CHUNK


def _gather_body(ids_hbm, table_hbm, out_hbm, idx_v, buf_a, buf_b, sem_g, sem_s):
    wid = lax.axis_index("s") * 2 + lax.axis_index("c")
    base = wid * _IDS_PER_W
    pltpu.sync_copy(ids_hbm.at[wid], idx_v)

    bufs = [buf_a, buf_b]
    gathers = [None] * _NCHUNK
    stores = [None] * _NCHUNK
    gathers[0] = pltpu.async_copy(table_hbm.at[idx_v.at[0]], bufs[0], sem_g)
    for c in range(_NCHUNK):
        if c + 1 < _NCHUNK:
            if c >= 1:
                stores[c - 1].wait()
            gathers[c + 1] = pltpu.async_copy(
                table_hbm.at[idx_v.at[c + 1]], bufs[(c + 1) % 2], sem_g)
        gathers[c].wait()
        stores[c] = pltpu.async_copy(
            bufs[c % 2], out_hbm.at[pl.ds(base + c * _CHUNK, _CHUNK)], sem_s)
    stores[_NCHUNK - 2].wait()
    stores[_NCHUNK - 1].wait()


@functools.partial(jax.jit, static_argnums=())
def _sc_gather(ids_flat, table):
    mesh = plsc.VectorSubcoreMesh(core_axis_name="c", subcore_axis_name="s")
    run = pl.kernel(
        _gather_body,
        out_type=jax.ShapeDtypeStruct((_NIDS, _HID), jnp.float32),
        mesh=mesh,
        scratch_types=[
            pltpu.VMEM((_NCHUNK, _CHUNK), jnp.int32),
            pltpu.VMEM((_CHUNK, _HID), jnp.float32),
            pltpu.VMEM((_CHUNK, _HID), jnp.float32),
            pltpu.SemaphoreType.DMA,
            pltpu.SemaphoreType.DMA,
        ],
    )
    return run(ids_flat.reshape(_NW, _NCHUNK, _CHUNK), table)


_NPE = 2 * _QLEN
_HALF = _HID // 2
_UR2 = 128

_INV_FREQ = np.exp(np.arange(_HALF, dtype=np.float64)
                   * (-math.log(10000.0) / _HALF)).astype(np.float32)


_NGRP = _NPE * _BSZ // 8
_NCH = 16
_CH = _NGRP // _NCH
_RING = 4


def _pos_chunk(invf_ref, c):
    u = lax.broadcasted_iota(jnp.int32, (_CH, _HALF), 0).astype(jnp.float32)
    inv_freq = jnp.broadcast_to(invf_ref[0:1, :], (_CH, _HALF))
    s_even = (_QLEN - 2 * _CH * c) - 2.0 * u
    ang_e = s_even * inv_freq
    ang_o = ang_e - inv_freq
    pe_e = jnp.concatenate([jnp.sin(ang_e), jnp.cos(ang_e)], axis=1)
    pe_o = jnp.concatenate([jnp.sin(ang_o), jnp.cos(ang_o)], axis=1)
    sub = lax.broadcasted_iota(jnp.int32, (_CH, 8, _HID), 1)
    return jnp.where(sub < 4, pe_e[:, None, :], pe_o[:, None, :])


def _pos_body(invf_ref, o_hbm, *scratch):
    bufs, sems = scratch[:_RING], scratch[_RING:]
    copies = [None] * _NCH
    for c in range(_NCH):
        b = c % _RING
        if c >= _RING:
            copies[c - _RING].wait()
        bufs[b][...] = _pos_chunk(invf_ref, c)
        copies[c] = pltpu.make_async_copy(
            bufs[b], o_hbm.at[pl.ds(c * _CH, _CH)], sems[b])
        copies[c].start()
    for c in range(_NCH - _RING, _NCH):
        copies[c].wait()


def _pos_emb():
    invf = jnp.asarray(np.broadcast_to(_INV_FREQ[None, :], (8, _HALF)))
    out = pl.pallas_call(
        _pos_body,
        out_shape=jax.ShapeDtypeStruct((_NGRP, 8, _HID), jnp.float32),
        in_specs=[pl.BlockSpec(memory_space=pltpu.MemorySpace.VMEM)],
        out_specs=pl.BlockSpec(memory_space=pltpu.MemorySpace.HBM),
        scratch_shapes=([pltpu.VMEM((_CH, 8, _HID), jnp.float32)] * _RING
                        + [pltpu.SemaphoreType.DMA] * _RING),
    )(invf)
    return out.reshape(_BSZ, 2 * _QLEN, _HID)


def kernel(token_id_input, W):
    word = _sc_gather(token_id_input.reshape(-1), W)
    pos = _pos_emb()
    return (word.reshape(_BSZ, _QLEN, _HID), pos)


# speedup vs baseline: 2.1821x; 1.1238x over previous
import functools
import math

import jax
import jax.numpy as jnp
import numpy as np
from jax import lax
from jax.experimental import pallas as pl
from jax.experimental.pallas import tpu as pltpu
from jax.experimental.pallas import tpu_sc as plsc

_VOCAB = 100000
_HID = 1024
_BSZ = 4
_QLEN = 2048
_NIDS = _BSZ * _QLEN
_NW = 32
_IDS_PER_W = _NIDS // _NW
_CHUNK = 32
_NCHUNK = _IDS_PER_W // _CHUNK


_SC_RING = 3


def _gather_body(ids_hbm, table_hbm, out_hbm, idx_v, buf_a, buf_b, buf_c,
                 sem_g, sem_s):
    wid = lax.axis_index("s") * 2 + lax.axis_index("c")
    base = wid * _IDS_PER_W
    pltpu.sync_copy(ids_hbm.at[wid], idx_v)

    bufs = [buf_a, buf_b, buf_c]
    gathers = [None] * _NCHUNK
    stores = [None] * _NCHUNK
    for c in range(_NCHUNK + _SC_RING - 1):
        if c < _NCHUNK:
            if c >= _SC_RING:
                stores[c - _SC_RING].wait()
            gathers[c] = pltpu.async_copy(
                table_hbm.at[idx_v.at[c]], bufs[c % _SC_RING], sem_g)
        j = c - (_SC_RING - 1)
        if 0 <= j < _NCHUNK:
            gathers[j].wait()
            stores[j] = pltpu.async_copy(
                bufs[j % _SC_RING],
                out_hbm.at[pl.ds(base + j * _CHUNK, _CHUNK)], sem_s)
    for j in range(_NCHUNK - _SC_RING, _NCHUNK):
        stores[j].wait()


@functools.partial(jax.jit, static_argnums=())
def _sc_gather(ids_flat, table):
    mesh = plsc.VectorSubcoreMesh(core_axis_name="c", subcore_axis_name="s")
    run = pl.kernel(
        _gather_body,
        out_type=jax.ShapeDtypeStruct((_NIDS, _HID), jnp.float32),
        mesh=mesh,
        scratch_types=[
            pltpu.VMEM((_NCHUNK, _CHUNK), jnp.int32),
            pltpu.VMEM((_CHUNK, _HID), jnp.float32),
            pltpu.VMEM((_CHUNK, _HID), jnp.float32),
            pltpu.VMEM((_CHUNK, _HID), jnp.float32),
            pltpu.SemaphoreType.DMA,
            pltpu.SemaphoreType.DMA,
        ],
    )
    return run(ids_flat.reshape(_NW, _NCHUNK, _CHUNK), table)


_NPE = 2 * _QLEN
_HALF = _HID // 2
_UR2 = 128

_INV_FREQ = np.exp(np.arange(_HALF, dtype=np.float64)
                   * (-math.log(10000.0) / _HALF)).astype(np.float32)


_NGRP = _NPE * _BSZ // 8
_NCH = 16
_CH = _NGRP // _NCH
_RING = 4


_TWO_OVER_PI = 0.6366197723675814
_P1 = 1.5707855224609375
_P2 = 1.0804334124e-05
_P3 = 1.9893587e-11
_SS = (-1.6666667163e-01, 8.3333337680e-03, -1.9841270114e-04, 2.7557314297e-06)
_CC = (-0.5, 4.1666667908e-02, -1.3888889225e-03, 2.4801587642e-05)


def _sincos(x):
    t = x * _TWO_OVER_PI
    q_i = (t + jnp.where(t >= 0, 0.5, -0.5)).astype(jnp.int32)
    k = q_i.astype(jnp.float32)
    r = ((x - k * _P1) - k * _P2) - k * _P3
    r2 = r * r
    sp = r + r * (r2 * (_SS[0] + r2 * (_SS[1] + r2 * (_SS[2] + r2 * _SS[3]))))
    cp = 1.0 + r2 * (_CC[0] + r2 * (_CC[1] + r2 * (_CC[2] + r2 * _CC[3])))
    swap = (q_i & 1) == 1
    q = q_i & 3
    neg_s = (q == 2) | (q == 3)
    neg_c = (q == 1) | (q == 2)
    sin_base = jnp.where(swap, cp, sp)
    cos_base = jnp.where(swap, sp, cp)
    sin_x = jnp.where(neg_s, -sin_base, sin_base)
    cos_x = jnp.where(neg_c, -cos_base, cos_base)
    return sin_x, cos_x


def _pos_chunk(invf_ref, c):
    u = lax.broadcasted_iota(jnp.int32, (_CH, _HALF), 0).astype(jnp.float32)
    inv_freq = jnp.broadcast_to(invf_ref[0:1, :], (_CH, _HALF))
    s_even = (_QLEN - 2 * _CH * c) - 2.0 * u
    ang_e = s_even * inv_freq
    ang_o = ang_e - inv_freq
    se, ce = _sincos(ang_e)
    so, co = _sincos(ang_o)
    pe_e = jnp.concatenate([se, ce], axis=1)
    pe_o = jnp.concatenate([so, co], axis=1)
    sub = lax.broadcasted_iota(jnp.int32, (_CH, 8, _HID), 1)
    return jnp.where(sub < 4, pe_e[:, None, :], pe_o[:, None, :])


def _pos_body(invf_ref, o_hbm, *scratch):
    bufs, sems = scratch[:_RING], scratch[_RING:]
    copies = [None] * _NCH
    for c in range(_NCH):
        b = c % _RING
        if c >= _RING:
            copies[c - _RING].wait()
        bufs[b][...] = _pos_chunk(invf_ref, c)
        copies[c] = pltpu.make_async_copy(
            bufs[b], o_hbm.at[pl.ds(c * _CH, _CH)], sems[b])
        copies[c].start()
    for c in range(_NCH - _RING, _NCH):
        copies[c].wait()


def _pos_emb():
    invf = jnp.asarray(np.broadcast_to(_INV_FREQ[None, :], (8, _HALF)))
    out = pl.pallas_call(
        _pos_body,
        out_shape=jax.ShapeDtypeStruct((_NGRP, 8, _HID), jnp.float32),
        in_specs=[pl.BlockSpec(memory_space=pltpu.MemorySpace.VMEM)],
        out_specs=pl.BlockSpec(memory_space=pltpu.MemorySpace.HBM),
        scratch_shapes=([pltpu.VMEM((_CH, 8, _HID), jnp.float32)] * _RING
                        + [pltpu.SemaphoreType.DMA] * _RING),
    )(invf)
    return out.reshape(_BSZ, 2 * _QLEN, _HID)


def kernel(token_id_input, W):
    word = _sc_gather(token_id_input.reshape(-1), W)
    pos = _pos_emb()
    return (word.reshape(_BSZ, _QLEN, _HID), pos)
